# Initial kernel scaffold; baseline (speedup 1.0000x reference)
import jax
import jax.numpy as jnp
from jax.experimental import pallas as pl


def kernel(x, prediction_edges, message_edges, message_edgewt, pool1_w, pool1_b, coef1, lin1_w, lin1_b, pool2_w, pool2_b, coef2, lin2_w, lin2_b, ewp_w, ewp_b, ep_w, ep_b):
    raise NotImplementedError("write your pallas kernel here")


# speedup vs baseline: 1.0247x; 1.0247x over previous
import functools

import jax
import jax.numpy as jnp
from jax import lax
from jax.experimental import pallas as pl
from jax.experimental.pallas import tpu as pltpu

N = 10000
D = 128
BLK = 2000


def _mm_kernel(x_ref, w_ref, o_ref):
    o_ref[...] = jnp.dot(x_ref[...], w_ref[...],
                         preferred_element_type=jnp.float32, precision=jax.lax.Precision.HIGHEST)


def _matmul(x, w):
    n, d = x.shape
    return pl.pallas_call(
        _mm_kernel,
        grid=(n // BLK,),
        in_specs=[pl.BlockSpec((BLK, d), lambda i: (i, 0)),
                  pl.BlockSpec((d, w.shape[1]), lambda i: (0, 0))],
        out_specs=pl.BlockSpec((BLK, w.shape[1]), lambda i: (i, 0)),
        out_shape=jax.ShapeDtypeStruct((n, w.shape[1]), jnp.float32),
    )(x, w)


def _layer_kernel(x_ref, m_ref, pb_ref, wa_ref, wb_ref, lb_ref, w2_ref,
                  h_ref, hp_ref):
    agg = jnp.maximum(m_ref[...] + pb_ref[...], 0.0)
    h = jnp.dot(x_ref[...], wa_ref[...], preferred_element_type=jnp.float32, precision=jax.lax.Precision.HIGHEST)
    h += jnp.dot(agg, wb_ref[...], preferred_element_type=jnp.float32, precision=jax.lax.Precision.HIGHEST)
    h = jnp.maximum(h + lb_ref[...], 0.0)
    h_ref[...] = h
    hp_ref[...] = jnp.dot(h, w2_ref[...], preferred_element_type=jnp.float32, precision=jax.lax.Precision.HIGHEST)


def _layer(x, m, pool_b, lin_w, lin_b, w2):
    n = x.shape[0]
    d2 = w2.shape[1]
    wa, wb = lin_w[:D], lin_w[D:]
    return pl.pallas_call(
        _layer_kernel,
        grid=(n // BLK,),
        in_specs=[pl.BlockSpec((BLK, D), lambda i: (i, 0)),
                  pl.BlockSpec((BLK, D), lambda i: (i, 0)),
                  pl.BlockSpec((1, D), lambda i: (0, 0)),
                  pl.BlockSpec((D, D), lambda i: (0, 0)),
                  pl.BlockSpec((D, D), lambda i: (0, 0)),
                  pl.BlockSpec((1, D), lambda i: (0, 0)),
                  pl.BlockSpec((D, d2), lambda i: (0, 0))],
        out_specs=[pl.BlockSpec((BLK, D), lambda i: (i, 0)),
                   pl.BlockSpec((BLK, d2), lambda i: (i, 0))],
        out_shape=[jax.ShapeDtypeStruct((n, D), jnp.float32),
                   jax.ShapeDtypeStruct((n, d2), jnp.float32)],
    )(x, m, pool_b.reshape(1, D), wa, wb, lin_b.reshape(1, D), w2)


def _segmax(xp, src, dst, scale, n):
    z = xp[src] * scale[:, None]
    return jax.ops.segment_max(z, dst, num_segments=n)


def kernel(x, prediction_edges, message_edges, message_edgewt,
           pool1_w, pool1_b, coef1, lin1_w, lin1_b,
           pool2_w, pool2_b, coef2, lin2_w, lin2_b,
           ewp_w, ewp_b, ep_w, ep_b):
    src, dst = message_edges[0], message_edges[1]
    n = x.shape[0]
    s1 = 1.0 + coef1 * message_edgewt
    s2 = 1.0 + coef2 * message_edgewt

    xp1 = _matmul(x, pool1_w)
    m1 = _segmax(xp1, src, dst, s1, n)
    h1, xp2 = _layer(x, m1, pool1_b, lin1_w, lin1_b, pool2_w)
    m2 = _segmax(xp2, src, dst, s2, n)
    wcat = jnp.concatenate([ewp_w, ep_w], axis=1)
    wcat = jnp.pad(wcat, ((0, 0), (0, 126)))
    h2, g = _layer(h1, m2, pool2_b, lin2_w, lin2_b, wcat)

    ee = g[prediction_edges[0]] + g[prediction_edges[1]]
    edge_weights = jax.nn.relu(ee[:, 0:1] + ewp_b)
    edge_predictor = ee[:, 1:2] + ep_b
    return (edge_weights, edge_predictor)
